# baseline (speedup 1.0000x reference)
import functools

import jax
import jax.numpy as jnp
from jax import lax
from jax.experimental import pallas as pl
from jax.experimental.pallas import tpu as pltpu
from jax.experimental.pallas import tpu_sc as plsc

VOCAB = 100000
EMBED_DIM = 64
BATCH = 4096
SEQ_LEN = 50

NUM_WORKERS = 32
BPW = BATCH // NUM_WORKERS
NBUF = 5
NGROUP = SEQ_LEN // NBUF

_MESH = plsc.VectorSubcoreMesh(core_axis_name="c", subcore_axis_name="s")


@functools.partial(
    pl.kernel,
    out_type=jax.ShapeDtypeStruct(
        (SEQ_LEN, EMBED_DIM // 8, NUM_WORKERS, 8, BPW), jnp.float32),
    mesh=_MESH,
    scratch_types=[
        pltpu.VMEM((SEQ_LEN, BPW), jnp.int32),
        pltpu.VMEM((NBUF, BPW, EMBED_DIM), jnp.float32),
        pltpu.VMEM((NBUF, EMBED_DIM // 8, 8, BPW), jnp.float32),
    ]
    + [pltpu.SemaphoreType.DMA] * (2 * NBUF),
    compiler_params=pltpu.CompilerParams(use_tc_tiling_on_sc=False,
                                         needs_layout_passes=False),
)
def _gather_kernel(idx_hbm, table_hbm, out_hbm, idx_v, rows_v, t_v, *sems):
    gsems, ssems = sems[:NBUF], sems[NBUF:]
    wid = lax.axis_index("s") * 2 + lax.axis_index("c")
    b0 = wid * BPW
    pltpu.sync_copy(idx_hbm.at[:, pl.ds(b0, BPW)], idx_v)

    lane = lax.iota(jnp.int32, 16)
    row_ids = [lane + 16 * g for g in range(BPW // 16)]

    def g_copy(s, b):
        return pltpu.make_async_copy(
            table_hbm.at[idx_v.at[s]], rows_v.at[b], gsems[b])

    def s_copy(s, b):
        return pltpu.make_async_copy(
            t_v.at[b], out_hbm.at[s, :, wid], ssems[b])

    def transpose(b):
        rb, tb = rows_v.at[b], t_v.at[b]

        @plsc.parallel_loop(0, EMBED_DIM, unroll=4)
        def c_body(c):
            d0 = c & 15
            colvec = (c - d0) + ((d0 + lane) & 15)
            dt, dr = colvec >> 3, colvec & 7
            for g in range(BPW // 16):
                vals = plsc.load_gather(rb, [row_ids[g], colvec])
                plsc.store_scatter(tb, [dt, dr, row_ids[g]], vals)

    for b in range(NBUF):
        g_copy(b, b).start()

    for b in range(NBUF):
        g_copy(b, b).wait()
        transpose(b)
        g_copy(NBUF + b, b).start()
        s_copy(b, b).start()

    def group_body(g):
        s0 = g * NBUF
        for b in range(NBUF):
            s = s0 + b
            g_copy(s, b).wait()
            s_copy(s - NBUF, b).wait()
            transpose(b)
            g_copy(s + NBUF, b).start()
            s_copy(s, b).start()

    pl.loop(1, NGROUP - 1)(group_body)

    s0 = SEQ_LEN - NBUF
    for b in range(NBUF):
        g_copy(s0 + b, b).wait()
        s_copy(s0 - NBUF + b, b).wait()
        transpose(b)
        s_copy(s0 + b, b).start()
    for b in range(NBUF):
        s_copy(s0 + b, b).wait()


def kernel(inputs, table):
    idx_t = inputs.T
    out = _gather_kernel(idx_t, table)
    return out.transpose(2, 4, 0, 1, 3).reshape(BATCH, SEQ_LEN, EMBED_DIM)


# speedup vs baseline: 1.0202x; 1.0202x over previous
import functools

import jax
import jax.numpy as jnp
from jax import lax
from jax.experimental import pallas as pl
from jax.experimental.pallas import tpu as pltpu
from jax.experimental.pallas import tpu_sc as plsc

VOCAB = 100000
EMBED_DIM = 64
BATCH = 4096
SEQ_LEN = 50

NUM_WORKERS = 32
BPW = BATCH // NUM_WORKERS
NBUF = 5
NGROUP = SEQ_LEN // NBUF

_MESH = plsc.VectorSubcoreMesh(core_axis_name="c", subcore_axis_name="s")


@functools.partial(
    pl.kernel,
    out_type=jax.ShapeDtypeStruct(
        (SEQ_LEN, EMBED_DIM // 8, NUM_WORKERS, 8, BPW), jnp.float32),
    mesh=_MESH,
    scratch_types=[
        pltpu.VMEM((SEQ_LEN, BPW), jnp.int32),
        pltpu.VMEM((NBUF, BPW, EMBED_DIM), jnp.float32),
        pltpu.VMEM((NBUF, EMBED_DIM // 8, 8, BPW), jnp.float32),
    ]
    + [pltpu.SemaphoreType.DMA] * (2 * NBUF),
    compiler_params=pltpu.CompilerParams(use_tc_tiling_on_sc=False,
                                         needs_layout_passes=False),
)
def _gather_kernel(idx_hbm, table_hbm, out_hbm, idx_v, rows_v, t_v, *sems):
    gsems, ssems = sems[:NBUF], sems[NBUF:]
    wid = lax.axis_index("s") * 2 + lax.axis_index("c")
    b0 = wid * BPW
    pltpu.sync_copy(idx_hbm.at[:, pl.ds(b0, BPW)], idx_v)

    lane = lax.iota(jnp.int32, 16)
    row_ids = [lane + 16 * g for g in range(BPW // 16)]

    def g_copy(s, b):
        return pltpu.make_async_copy(
            table_hbm.at[idx_v.at[s]], rows_v.at[b], gsems[b])

    def s_copy(s, b):
        return pltpu.make_async_copy(
            t_v.at[b], out_hbm.at[s, :, wid], ssems[b])

    def transpose(b):
        rb, tb = rows_v.at[b], t_v.at[b]

        @plsc.parallel_loop(0, EMBED_DIM, unroll=2)
        def c_body(c):
            d0 = c & 15
            colvec = (c - d0) + ((d0 + lane) & 15)
            dt, dr = colvec >> 3, colvec & 7
            for g in range(BPW // 16):
                vals = plsc.load_gather(rb, [row_ids[g], colvec])
                plsc.store_scatter(tb, [dt, dr, row_ids[g]], vals)

    for b in range(NBUF):
        g_copy(b, b).start()

    for b in range(NBUF):
        g_copy(b, b).wait()
        transpose(b)
        g_copy(NBUF + b, b).start()
        s_copy(b, b).start()

    def group_body(g):
        s0 = g * NBUF
        for b in range(NBUF):
            s = s0 + b
            g_copy(s, b).wait()
            s_copy(s - NBUF, b).wait()
            transpose(b)
            g_copy(s + NBUF, b).start()
            s_copy(s, b).start()

    pl.loop(1, NGROUP - 1)(group_body)

    s0 = SEQ_LEN - NBUF
    for b in range(NBUF):
        g_copy(s0 + b, b).wait()
        s_copy(s0 - NBUF + b, b).wait()
        transpose(b)
        s_copy(s0 + b, b).start()
    for b in range(NBUF):
        s_copy(s0 + b, b).wait()


def kernel(inputs, table):
    idx_t = inputs.T
    out = _gather_kernel(idx_t, table)
    return out.transpose(2, 4, 0, 1, 3).reshape(BATCH, SEQ_LEN, EMBED_DIM)
